# Initial kernel scaffold; baseline (speedup 1.0000x reference)
import jax
import jax.numpy as jnp
from jax.experimental import pallas as pl


def kernel(input_ids, main_table, hooked_table):
    raise NotImplementedError("write your pallas kernel here")


# speedup vs baseline: 6.0905x; 6.0905x over previous
import functools

import jax
import jax.numpy as jnp
from jax import lax
from jax.experimental import pallas as pl
from jax.experimental.pallas import tpu as pltpu
from jax.experimental.pallas import tpu_sc as plsc

HOOK = 42
LANES = 16
IDX_PER_DMA = 128
DMAS_PER_CHUNK = 4
CHUNK = IDX_PER_DMA * DMAS_PER_CHUNK
NBUF = 2


def _body(n_chunks, ids_hbm, table_hbm, hooked_hbm, out_hbm,
          idx0, idx1, rows0, rows1, hrep, sg0, sg1, sw0, sw1):
  n_cores = 2
  cid = lax.axis_index("c")
  sid = lax.axis_index("s")
  w = sid * n_cores + cid
  D = hrep.shape[1]
  idx_rows_per_worker = n_chunks * DMAS_PER_CHUNK

  pltpu.sync_copy(hooked_hbm.at[0], hrep.at[0])
  hvecs = [hrep[0, pl.ds(c * LANES, LANES)] for c in range(D // LANES)]

  idx = (idx0, idx1)
  rows = (rows0, rows1)
  sg = (sg0, sg1)
  sw = (sw0, sw1)

  def stage_and_fire(chunk_i, b):
    row0 = w * idx_rows_per_worker + chunk_i * DMAS_PER_CHUNK
    pltpu.sync_copy(ids_hbm.at[pl.ds(row0, DMAS_PER_CHUNK)], idx[b])
    for j in range(DMAS_PER_CHUNK):
      pltpu.async_copy(table_hbm.at[idx[b].at[j]],
                       rows[b].at[pl.ds(j * IDX_PER_DMA, IDX_PER_DMA)],
                       sg[b])

  def drain_gathers(b):
    for j in range(DMAS_PER_CHUNK):
      pltpu.make_async_copy(table_hbm.at[idx[b].at[j]],
                            rows[b].at[pl.ds(j * IDX_PER_DMA, IDX_PER_DMA)],
                            sg[b]).wait()

  def fixup(b):
    acc = jnp.zeros((LANES,), jnp.bool_)
    for j in range(DMAS_PER_CHUNK):
      for k in range(IDX_PER_DMA // LANES):
        v = idx[b][j, pl.ds(k * LANES, LANES)]
        acc = acc | (v == HOOK)

    @pl.when(plsc.all_reduce_population_count(acc)[0] > 0)
    def _rare():
      def patch(s2, carry):
        j = s2 // (IDX_PER_DMA // LANES)
        k = s2 % (IDX_PER_DMA // LANES)
        v = idx[b][j, pl.ds(k * LANES, LANES)]

        @pl.when(plsc.all_reduce_population_count(v == HOOK)[0] > 0)
        def _subvec():
          for l in range(LANES):
            vi = v[l]

            @pl.when(vi == HOOK)
            def _lane():
              r = j * IDX_PER_DMA + k * LANES + l
              for c in range(len(hvecs)):
                rows[b][r, pl.ds(c * LANES, LANES)] = hvecs[c]
        return carry
      lax.fori_loop(0, CHUNK // LANES, patch, 0)

  def out_base(chunk_i):
    return w * (n_chunks * CHUNK) + chunk_i * CHUNK

  def fire_outwrite(chunk_i, b):
    pltpu.async_copy(rows[b].at[pl.ds(0, CHUNK)],
                     out_hbm.at[pl.ds(out_base(chunk_i), CHUNK)],
                     sw[b])

  def wait_outwrite(b):
    pltpu.make_async_copy(rows[b].at[pl.ds(0, CHUNK)],
                          out_hbm.at[pl.ds(0, CHUNK)],
                          sw[b]).wait()

  for b in range(NBUF):
    stage_and_fire(b, b)

  def step(s, carry):
    for b in range(NBUF):
      i = s * NBUF + b
      drain_gathers(b)
      fixup(b)
      fire_outwrite(i, b)

      @pl.when(s < n_chunks // NBUF - 1)
      def _prefetch():
        wait_outwrite(b)
        stage_and_fire(i + NBUF, b)
    return carry

  lax.fori_loop(0, n_chunks // NBUF, step, 0)

  for b in range(NBUF):
    wait_outwrite(b)


def kernel(input_ids, main_table, hooked_table):
  B, L = input_ids.shape
  V, D = main_table.shape
  N = B * L
  n_workers = 32
  assert N % (n_workers * CHUNK * NBUF) == 0
  n_chunks = N // (n_workers * CHUNK)

  ids = input_ids.reshape(N // IDX_PER_DMA, IDX_PER_DMA).astype(jnp.int32)

  mesh = plsc.VectorSubcoreMesh(core_axis_name="c", subcore_axis_name="s")
  run = pl.kernel(
      functools.partial(_body, n_chunks),
      out_type=jax.ShapeDtypeStruct((N, D), jnp.float32),
      mesh=mesh,
      compiler_params=pltpu.CompilerParams(
          needs_layout_passes=False, use_tc_tiling_on_sc=False),
      scratch_types=[
          pltpu.VMEM((DMAS_PER_CHUNK, IDX_PER_DMA), jnp.int32),
          pltpu.VMEM((DMAS_PER_CHUNK, IDX_PER_DMA), jnp.int32),
          pltpu.VMEM((CHUNK + 1, D), jnp.float32),
          pltpu.VMEM((CHUNK + 1, D), jnp.float32),
          pltpu.VMEM((1, D), jnp.float32),
          pltpu.SemaphoreType.DMA,
          pltpu.SemaphoreType.DMA,
          pltpu.SemaphoreType.DMA,
          pltpu.SemaphoreType.DMA,
      ],
  )
  out = run(ids, main_table, hooked_table.astype(jnp.float32))
  return out.reshape(B, L, D)
